# baseline (speedup 1.0000x reference)
import jax
import jax.numpy as jnp
from jax import lax
from jax.experimental import pallas as pl
from jax.experimental.pallas import tpu as pltpu
from jax.experimental.pallas import tpu_sc as plsc

N = 10000
D = 128
E = 320000
PE = 100000

NC, NS = 2, 16
NW = NC * NS
CH = 112
NCH = 92
EPAD = NW * NCH * CH
NPAD = NS * 640
RPT = NPAD // NS
NB = 3
RD = 5

PCH = 25
PW = 128
PPAD = NW * PCH * PW

_mesh = lambda: plsc.VectorSubcoreMesh(core_axis_name="c", subcore_axis_name="s")


def _make_seg(want_deg):
  out_type = [jax.ShapeDtypeStruct((NC, NPAD, D), jnp.float32)]
  scratch = [
      pltpu.VMEM((RD, 2, CH), jnp.int32),
      pltpu.VMEM((RD, CH), jnp.float32),
      pltpu.VMEM((CH, D), jnp.float32),
      pltpu.VMEM((CH, D), jnp.float32),
      pltpu.VMEM((CH, D), jnp.float32),
      pltpu.VMEM_SHARED((NPAD, D), jnp.float32),
  ] + [pltpu.SemaphoreType.DMA] * 9
  if want_deg:
    out_type.append(jax.ShapeDtypeStruct((NC, NPAD), jnp.float32))
    scratch += [
        pltpu.VMEM((NB, CH), jnp.float32),
        pltpu.VMEM((RPT,), jnp.float32),
        pltpu.VMEM_SHARED((NPAD,), jnp.float32),
    ]

  def body(x_hbm, e_hbm, w_hbm, *rest):
    if want_deg:
      (out_hbm, deg_hbm, ed, wring, rows0, rows1, rows2, acc,
       gs0, gs1, gs2, ss0, ss1, ss2, fs0, fs1, fs2,
       abuf, zbuf, dacc) = rest
    else:
      (out_hbm, ed, wring, rows0, rows1, rows2, acc,
       gs0, gs1, gs2, ss0, ss1, ss2, fs0, fs1, fs2) = rest
      abuf = zbuf = dacc = None
    cid = lax.axis_index("c")
    sid = lax.axis_index("s")
    wid = cid * NS + sid
    rowsb = (rows0, rows1, rows2)
    gs = (gs0, gs1, gs2)
    ss = (ss0, ss1, ss2)
    fs = (fs0, fs1, fs2)

    def zrow(i, carry):
      for j in range(D // 16):
        rows0[i, pl.ds(j * 16, 16)] = jnp.zeros((16,), jnp.float32)
      return carry
    lax.fori_loop(0, CH, zrow, 0)
    for k in range(RPT // CH):
      pltpu.sync_copy(rows0, acc.at[pl.ds(sid * RPT + k * CH, CH)])
    if RPT % CH:
      pltpu.sync_copy(rows0.at[pl.ds(0, RPT % CH)],
                      acc.at[pl.ds(sid * RPT + (RPT // CH) * CH, RPT % CH)])
    if want_deg:
      def zd(i, carry):
        zbuf[pl.ds(i * 16, 16)] = jnp.zeros((16,), jnp.float32)
        return carry
      lax.fori_loop(0, RPT // 16, zd, 0)
      pltpu.sync_copy(zbuf, dacc.at[pl.ds(sid * RPT, RPT)])
    plsc.subcore_barrier()

    def slot(c):
      return c % RD if isinstance(c, int) else lax.rem(c, RD)

    def fill(c, s):
      pltpu.async_copy(e_hbm.at[wid, c], ed.at[slot(c)], fs[s])
      pltpu.async_copy(w_hbm.at[wid, c], wring.at[slot(c)], fs[s])

    def f_wait(c, s):
      pltpu.make_async_copy(e_hbm.at[wid, c], ed.at[slot(c)], fs[s]).wait()
      pltpu.make_async_copy(w_hbm.at[wid, c], wring.at[slot(c)],
                            fs[s]).wait()

    def g_issue(c, b):
      pltpu.async_copy(x_hbm.at[ed.at[slot(c), 0]], rowsb[b], gs[b])

    def g_wait(c, b):
      pltpu.make_async_copy(x_hbm.at[ed.at[slot(c), 0]], rowsb[b],
                            gs[b]).wait()

    def scale(c, b):
      rows = rowsb[b]
      r5 = slot(c)
      def egroup(g, carry):
        wr16 = wring[r5, pl.ds(g * 16, 16)]
        for r in range(16):
          ws = wr16[r]
          for j in range(D // 16):
            sl = pl.ds(j * 16, 16)
            rows[g * 16 + r, sl] = rows[g * 16 + r, sl] * ws
        return carry
      lax.fori_loop(0, CH // 16, egroup, 0)

    def s_issue(c, b):
      r5 = slot(c)
      pltpu.async_copy(rowsb[b], acc.at[ed.at[r5, 1]], ss[b], add=True)
      if want_deg:
        for j in range(CH // 16):
          sl = pl.ds(j * 16, 16)
          abuf[b, sl] = jnp.abs(wring[r5, sl])
        pltpu.async_copy(abuf.at[b], dacc.at[ed.at[r5, 1]], ss[b], add=True)

    def s_wait(c, b):
      r5 = slot(c)
      pltpu.make_async_copy(rowsb[b], acc.at[ed.at[r5, 1]], ss[b]).wait()
      if want_deg:
        pltpu.make_async_copy(abuf.at[b], dacc.at[ed.at[r5, 1]], ss[b]).wait()

    fill(0, 0)
    fill(1, 1)
    fill(2, 2)
    f_wait(0, 0)
    g_issue(0, 0)
    g_wait(0, 0); scale(0, 0); s_issue(0, 0)
    fill(3, 0)
    f_wait(1, 1); g_issue(1, 1)
    g_wait(1, 1); scale(1, 1); s_issue(1, 1)
    fill(4, 1)
    f_wait(2, 2); g_issue(2, 2)

    def step(i, carry):
      for k in range(3):
        c = 3 * i + k + 2
        b = (k + 2) % 3
        g_wait(c, b)
        scale(c, b)
        s_issue(c, b)
        s_wait(c - 2, (b + 1) % 3)
        @pl.when(c + 3 < NCH)
        def _():
          fill(c + 3, b)
        @pl.when(c + 1 < NCH)
        def _():
          f_wait(c + 1, (b + 1) % 3)
          g_issue(c + 1, (b + 1) % 3)
      return carry
    lax.fori_loop(0, (NCH - 2) // 3, step, 0)

    s_wait(NCH - 2, (NCH - 2) % 3)
    s_wait(NCH - 1, (NCH - 1) % 3)
    plsc.subcore_barrier()

    pltpu.sync_copy(acc.at[pl.ds(sid * RPT, RPT)],
                    out_hbm.at[cid].at[pl.ds(sid * RPT, RPT)])
    if want_deg:
      pltpu.sync_copy(dacc.at[pl.ds(sid * RPT, RPT)],
                      deg_hbm.at[cid].at[pl.ds(sid * RPT, RPT)])

  return pl.kernel(body, out_type=tuple(out_type) if want_deg else out_type[0],
                   mesh=_mesh(), scratch_types=scratch)


def _make_pred():
  out_type = (jax.ShapeDtypeStruct((NW, PCH, PW), jnp.float32),
              jax.ShapeDtypeStruct((NW, PCH, PW), jnp.float32))
  scratch = [
      pltpu.VMEM((PCH, PW), jnp.int32),
      pltpu.VMEM((PCH, PW), jnp.int32),
      pltpu.VMEM((PW,), jnp.float32),
      pltpu.VMEM((PW,), jnp.float32),
      pltpu.VMEM((PW,), jnp.float32),
      pltpu.VMEM((PW,), jnp.float32),
      pltpu.SemaphoreType.DMA,
  ]

  def body(p0_hbm, p1_hbm, p2_hbm, p3_hbm, ps_hbm, pd_hbm,
           o0_hbm, o1_hbm, siv, div, a0, a1, b0, b1, sem):
    cid = lax.axis_index("c")
    sid = lax.axis_index("s")
    wid = cid * NS + sid
    pltpu.sync_copy(ps_hbm.at[wid], siv)
    pltpu.sync_copy(pd_hbm.at[wid], div)

    def chunk(c, carry):
      cp0 = pltpu.async_copy(p0_hbm.at[siv.at[c]], a0, sem)
      cp1 = pltpu.async_copy(p1_hbm.at[siv.at[c]], a1, sem)
      cp2 = pltpu.async_copy(p2_hbm.at[div.at[c]], b0, sem)
      cp3 = pltpu.async_copy(p3_hbm.at[div.at[c]], b1, sem)
      cp0.wait(); cp1.wait(); cp2.wait(); cp3.wait()
      for j in range(PW // 16):
        sl = pl.ds(j * 16, 16)
        a0[sl] = a0[sl] + b0[sl]
        a1[sl] = a1[sl] + b1[sl]
      pltpu.sync_copy(a0, o0_hbm.at[wid, c])
      pltpu.sync_copy(a1, o1_hbm.at[wid, c])
      return carry
    lax.fori_loop(0, PCH, chunk, 0)

  return pl.kernel(body, out_type=out_type, mesh=_mesh(),
                   scratch_types=scratch)


_BLK = 1000
_GRID = N // _BLK


def _tc_layer1(x, aggp, deg0, deg1, Wa, Wb, b):
  def body(x_r, a0_r, a1_r, d0_r, d1_r, wa_r, wb_r, b_r, h_r, rd_r):
    rd = 1.0 / jnp.maximum(d0_r[...] + d1_r[...], 1.0)
    a = (a0_r[0] + a1_r[0]) * rd
    acc = jnp.dot(x_r[...], wa_r[...], preferred_element_type=jnp.float32)
    acc += jnp.dot(a, wb_r[...], preferred_element_type=jnp.float32)
    h_r[...] = jnp.tanh(acc + b_r[...])
    rd_r[...] = rd

  row = pl.BlockSpec((_BLK, D), lambda i: (i, 0))
  col1 = pl.BlockSpec((_BLK, 1), lambda i: (i, 0))
  p0 = pl.BlockSpec((1, _BLK, D), lambda i: (0, i, 0))
  p1 = pl.BlockSpec((1, _BLK, D), lambda i: (1, i, 0))
  full = pl.BlockSpec((D, D), lambda i: (0, 0))
  bias = pl.BlockSpec((1, D), lambda i: (0, 0))
  return pl.pallas_call(
      body,
      grid=(_GRID,),
      in_specs=[row, p0, p1, col1, col1, full, full, bias],
      out_specs=[row, col1],
      out_shape=[jax.ShapeDtypeStruct((N, D), jnp.float32),
                 jax.ShapeDtypeStruct((N, 1), jnp.float32)],
  )(x, aggp, aggp, deg0, deg1, Wa, Wb, b)


def _tc_layer2(h1, aggp, rdeg, Wa, Wb, b, Wp1, Wp2, bp4):
  def body(h1_r, a0_r, a1_r, rd_r, wa_r, wb_r, b_r, wp1_r, wp2_r, bp_r, p_r):
    a = (a0_r[0] + a1_r[0]) * rd_r[...]
    acc = jnp.dot(h1_r[...], wa_r[...], preferred_element_type=jnp.float32)
    acc += jnp.dot(a, wb_r[...], preferred_element_type=jnp.float32)
    h2 = jnp.tanh(acc + b_r[...])
    p = jnp.dot(h1_r[...], wp1_r[...], preferred_element_type=jnp.float32)
    p += jnp.dot(h2, wp2_r[...], preferred_element_type=jnp.float32)
    p_r[...] = p + bp_r[...]

  row = pl.BlockSpec((_BLK, D), lambda i: (i, 0))
  col1 = pl.BlockSpec((_BLK, 1), lambda i: (i, 0))
  p0 = pl.BlockSpec((1, _BLK, D), lambda i: (0, i, 0))
  p1 = pl.BlockSpec((1, _BLK, D), lambda i: (1, i, 0))
  full = pl.BlockSpec((D, D), lambda i: (0, 0))
  bias = pl.BlockSpec((1, D), lambda i: (0, 0))
  proj = pl.BlockSpec((D, 4), lambda i: (0, 0))
  bias4 = pl.BlockSpec((1, 4), lambda i: (0, 0))
  return pl.pallas_call(
      body,
      grid=(_GRID,),
      in_specs=[row, p0, p1, col1, full, full, bias, proj, proj, bias4],
      out_specs=pl.BlockSpec((_BLK, 4), lambda i: (i, 0)),
      out_shape=jax.ShapeDtypeStruct((N, 4), jnp.float32),
  )(h1, aggp, aggp, rdeg, Wa, Wb, b, Wp1, Wp2, bp4)


@jax.jit
def kernel(x, edge_index, edge_weight, pred_edge_index, W1, b1, W2, b2, Wp, bp):
  ew = edge_weight.astype(jnp.float32)
  src = jnp.concatenate([edge_index[0], jnp.zeros((EPAD - E,), jnp.int32)])
  dst = jnp.concatenate([edge_index[1], jnp.zeros((EPAD - E,), jnp.int32)])
  w = jnp.concatenate([ew, jnp.zeros((EPAD - E,), jnp.float32)])
  e3 = jnp.stack([src.reshape(NW, NCH, CH), dst.reshape(NW, NCH, CH)],
                 axis=2)
  w3 = w.reshape(NW, NCH, CH)

  agg1p, degp = _make_seg(True)(x, e3, w3)
  deg0 = degp[0, :N, None]
  deg1 = degp[1, :N, None]
  h1, rdeg = _tc_layer1(x, agg1p, deg0, deg1, W1[:D], W1[D:], b1[None, :])

  agg2p = _make_seg(False)(h1, e3, w3)
  Wp1 = jnp.concatenate([Wp[0:D], Wp[2 * D:3 * D]], axis=1)
  Wp2 = jnp.concatenate([Wp[D:2 * D], Wp[3 * D:]], axis=1)
  bp4 = jnp.concatenate([bp, jnp.zeros((2,), jnp.float32)])[None, :]
  P = _tc_layer2(h1, agg2p, rdeg, W2[:D], W2[D:], b2[None, :], Wp1, Wp2, bp4)

  ps = jnp.concatenate([pred_edge_index[0],
                        jnp.zeros((PPAD - PE,), jnp.int32)]).reshape(NW, PCH, PW)
  pd = jnp.concatenate([pred_edge_index[1],
                        jnp.zeros((PPAD - PE,), jnp.int32)]).reshape(NW, PCH, PW)
  p0 = P[:, 0] + 0.0
  p1 = P[:, 1] + 0.0
  p2 = P[:, 2] + 0.0
  p3 = P[:, 3] + 0.0
  o0, o1 = _make_pred()(p0, p1, p2, p3, ps, pd)
  return jnp.stack([o0.reshape(-1)[:PE], o1.reshape(-1)[:PE]], axis=1)


# speedup vs baseline: 1.8550x; 1.8550x over previous
import jax
import jax.numpy as jnp
from jax import lax
from jax.experimental import pallas as pl
from jax.experimental.pallas import tpu as pltpu
from jax.experimental.pallas import tpu_sc as plsc

N = 10000
D = 128
E = 320000
PE = 100000

NC, NS = 2, 16
NW = NC * NS
CH = 112
NCH = 90
EPAD = NW * NCH * CH
NPAD = NS * 640
RPT = NPAD // NS
NB = 3
RD = 5

PCH = 25
PW = 128
PPAD = NW * PCH * PW

_mesh = lambda: plsc.VectorSubcoreMesh(core_axis_name="c", subcore_axis_name="s")


def _make_seg(want_deg):
  out_type = [jax.ShapeDtypeStruct((NC, NPAD, D), jnp.float32)]
  scratch = [
      pltpu.VMEM((RD, 2, CH), jnp.int32),
      pltpu.VMEM((RD, CH), jnp.float32),
      pltpu.VMEM((CH, D), jnp.float32),
      pltpu.VMEM((CH, D), jnp.float32),
      pltpu.VMEM((CH, D), jnp.float32),
      pltpu.VMEM_SHARED((NPAD, D), jnp.float32),
  ] + [pltpu.SemaphoreType.DMA] * 9
  if want_deg:
    out_type.append(jax.ShapeDtypeStruct((NC, NPAD), jnp.float32))
    scratch += [
        pltpu.VMEM((NB, CH), jnp.float32),
        pltpu.VMEM((RPT,), jnp.float32),
        pltpu.VMEM_SHARED((NPAD,), jnp.float32),
    ]

  def body(x_hbm, e_hbm, w_hbm, *rest):
    if want_deg:
      (out_hbm, deg_hbm, ed, wring, rows0, rows1, rows2, acc,
       gs0, gs1, gs2, ss0, ss1, ss2, fs0, fs1, fs2,
       abuf, zbuf, dacc) = rest
    else:
      (out_hbm, ed, wring, rows0, rows1, rows2, acc,
       gs0, gs1, gs2, ss0, ss1, ss2, fs0, fs1, fs2) = rest
      abuf = zbuf = dacc = None
    cid = lax.axis_index("c")
    sid = lax.axis_index("s")
    wid = cid * NS + sid
    rowsb = (rows0, rows1, rows2)
    gs = (gs0, gs1, gs2)
    ss = (ss0, ss1, ss2)
    fs = (fs0, fs1, fs2)

    def zrow(i, carry):
      for j in range(D // 16):
        rows0[i, pl.ds(j * 16, 16)] = jnp.zeros((16,), jnp.float32)
      return carry
    lax.fori_loop(0, CH, zrow, 0)
    for k in range(RPT // CH):
      pltpu.sync_copy(rows0, acc.at[pl.ds(sid * RPT + k * CH, CH)])
    if RPT % CH:
      pltpu.sync_copy(rows0.at[pl.ds(0, RPT % CH)],
                      acc.at[pl.ds(sid * RPT + (RPT // CH) * CH, RPT % CH)])
    if want_deg:
      def zd(i, carry):
        zbuf[pl.ds(i * 16, 16)] = jnp.zeros((16,), jnp.float32)
        return carry
      lax.fori_loop(0, RPT // 16, zd, 0)
      pltpu.sync_copy(zbuf, dacc.at[pl.ds(sid * RPT, RPT)])
    plsc.subcore_barrier()

    def slot(c):
      return c % RD if isinstance(c, int) else lax.rem(c, RD)

    def fill(c, s):
      pltpu.async_copy(e_hbm.at[wid, c], ed.at[slot(c)], fs[s])
      pltpu.async_copy(w_hbm.at[wid, c], wring.at[slot(c)], fs[s])

    def f_wait(c, s):
      pltpu.make_async_copy(e_hbm.at[wid, c], ed.at[slot(c)], fs[s]).wait()
      pltpu.make_async_copy(w_hbm.at[wid, c], wring.at[slot(c)],
                            fs[s]).wait()

    def g_issue(c, b):
      pltpu.async_copy(x_hbm.at[ed.at[slot(c), 0]], rowsb[b], gs[b])

    def g_wait(c, b):
      pltpu.make_async_copy(x_hbm.at[ed.at[slot(c), 0]], rowsb[b],
                            gs[b]).wait()

    def scale(c, b):
      rows = rowsb[b]
      r5 = slot(c)
      def egroup(g, carry):
        wr16 = wring[r5, pl.ds(g * 16, 16)]
        for r in range(16):
          ws = wr16[r]
          for j in range(D // 16):
            sl = pl.ds(j * 16, 16)
            rows[g * 16 + r, sl] = rows[g * 16 + r, sl] * ws
        return carry
      lax.fori_loop(0, CH // 16, egroup, 0)

    def s_issue(c, b):
      r5 = slot(c)
      pltpu.async_copy(rowsb[b], acc.at[ed.at[r5, 1]], ss[b], add=True)
      if want_deg:
        for j in range(CH // 16):
          sl = pl.ds(j * 16, 16)
          abuf[b, sl] = jnp.abs(wring[r5, sl])
        pltpu.async_copy(abuf.at[b], dacc.at[ed.at[r5, 1]], ss[b], add=True)

    def s_wait(c, b):
      r5 = slot(c)
      pltpu.make_async_copy(rowsb[b], acc.at[ed.at[r5, 1]], ss[b]).wait()
      if want_deg:
        pltpu.make_async_copy(abuf.at[b], dacc.at[ed.at[r5, 1]], ss[b]).wait()

    fill(0, 0)
    fill(1, 1)
    fill(2, 2)
    f_wait(0, 0)
    g_issue(0, 0)
    g_wait(0, 0); scale(0, 0); s_issue(0, 0)
    fill(3, 0)
    f_wait(1, 1); g_issue(1, 1)
    g_wait(1, 1); scale(1, 1); s_issue(1, 1)
    fill(4, 1)
    f_wait(2, 2); g_issue(2, 2)

    def step(i, carry):
      for b in range(3):
        c = 3 * i + b
        g_wait(c, b)
        scale(c, b)
        s_issue(c, b)
        s_wait(c - 2, (b + 1) % 3)
        @pl.when(c + 3 < NCH)
        def _():
          fill(c + 3, b)
        @pl.when(c + 1 < NCH)
        def _():
          f_wait(c + 1, (b + 1) % 3)
          g_issue(c + 1, (b + 1) % 3)
      return carry
    g_wait(2, 2); scale(2, 2); s_issue(2, 2)
    s_wait(0, 0)
    fill(5, 2)
    f_wait(3, 0); g_issue(3, 0)
    lax.fori_loop(1, NCH // 3, step, 0)

    s_wait(NCH - 2, (NCH - 2) % 3)
    s_wait(NCH - 1, (NCH - 1) % 3)
    plsc.subcore_barrier()

    pltpu.sync_copy(acc.at[pl.ds(sid * RPT, RPT)],
                    out_hbm.at[cid].at[pl.ds(sid * RPT, RPT)])
    if want_deg:
      pltpu.sync_copy(dacc.at[pl.ds(sid * RPT, RPT)],
                      deg_hbm.at[cid].at[pl.ds(sid * RPT, RPT)])

  return pl.kernel(body, out_type=tuple(out_type) if want_deg else out_type[0],
                   mesh=_mesh(), scratch_types=scratch)


def _make_pred():
  out_type = (jax.ShapeDtypeStruct((NW, PCH, PW), jnp.float32),
              jax.ShapeDtypeStruct((NW, PCH, PW), jnp.float32))
  scratch = [
      pltpu.VMEM((PCH, PW), jnp.int32),
      pltpu.VMEM((PCH, PW), jnp.int32),
      pltpu.VMEM((PW,), jnp.float32),
      pltpu.VMEM((PW,), jnp.float32),
      pltpu.VMEM((PW,), jnp.float32),
      pltpu.VMEM((PW,), jnp.float32),
      pltpu.SemaphoreType.DMA,
  ]

  def body(p0_hbm, p1_hbm, p2_hbm, p3_hbm, ps_hbm, pd_hbm,
           o0_hbm, o1_hbm, siv, div, a0, a1, b0, b1, sem):
    cid = lax.axis_index("c")
    sid = lax.axis_index("s")
    wid = cid * NS + sid
    pltpu.sync_copy(ps_hbm.at[wid], siv)
    pltpu.sync_copy(pd_hbm.at[wid], div)

    def chunk(c, carry):
      cp0 = pltpu.async_copy(p0_hbm.at[siv.at[c]], a0, sem)
      cp1 = pltpu.async_copy(p1_hbm.at[siv.at[c]], a1, sem)
      cp2 = pltpu.async_copy(p2_hbm.at[div.at[c]], b0, sem)
      cp3 = pltpu.async_copy(p3_hbm.at[div.at[c]], b1, sem)
      cp0.wait(); cp1.wait(); cp2.wait(); cp3.wait()
      for j in range(PW // 16):
        sl = pl.ds(j * 16, 16)
        a0[sl] = a0[sl] + b0[sl]
        a1[sl] = a1[sl] + b1[sl]
      pltpu.sync_copy(a0, o0_hbm.at[wid, c])
      pltpu.sync_copy(a1, o1_hbm.at[wid, c])
      return carry
    lax.fori_loop(0, PCH, chunk, 0)

  return pl.kernel(body, out_type=out_type, mesh=_mesh(),
                   scratch_types=scratch)


_BLK = 1000
_GRID = N // _BLK


def _tc_layer1(x, aggp, deg0, deg1, Wa, Wb, b):
  def body(x_r, a0_r, a1_r, d0_r, d1_r, wa_r, wb_r, b_r, h_r, rd_r):
    rd = 1.0 / jnp.maximum(d0_r[...] + d1_r[...], 1.0)
    a = (a0_r[0] + a1_r[0]) * rd
    acc = jnp.dot(x_r[...], wa_r[...], preferred_element_type=jnp.float32)
    acc += jnp.dot(a, wb_r[...], preferred_element_type=jnp.float32)
    h_r[...] = jnp.tanh(acc + b_r[...])
    rd_r[...] = rd

  row = pl.BlockSpec((_BLK, D), lambda i: (i, 0))
  col1 = pl.BlockSpec((_BLK, 1), lambda i: (i, 0))
  p0 = pl.BlockSpec((1, _BLK, D), lambda i: (0, i, 0))
  p1 = pl.BlockSpec((1, _BLK, D), lambda i: (1, i, 0))
  full = pl.BlockSpec((D, D), lambda i: (0, 0))
  bias = pl.BlockSpec((1, D), lambda i: (0, 0))
  return pl.pallas_call(
      body,
      grid=(_GRID,),
      in_specs=[row, p0, p1, col1, col1, full, full, bias],
      out_specs=[row, col1],
      out_shape=[jax.ShapeDtypeStruct((N, D), jnp.float32),
                 jax.ShapeDtypeStruct((N, 1), jnp.float32)],
  )(x, aggp, aggp, deg0, deg1, Wa, Wb, b)


def _tc_layer2(h1, aggp, rdeg, Wa, Wb, b, Wp1, Wp2, bp4):
  def body(h1_r, a0_r, a1_r, rd_r, wa_r, wb_r, b_r, wp1_r, wp2_r, bp_r, p_r):
    a = (a0_r[0] + a1_r[0]) * rd_r[...]
    acc = jnp.dot(h1_r[...], wa_r[...], preferred_element_type=jnp.float32)
    acc += jnp.dot(a, wb_r[...], preferred_element_type=jnp.float32)
    h2 = jnp.tanh(acc + b_r[...])
    p = jnp.dot(h1_r[...], wp1_r[...], preferred_element_type=jnp.float32)
    p += jnp.dot(h2, wp2_r[...], preferred_element_type=jnp.float32)
    p_r[...] = p + bp_r[...]

  row = pl.BlockSpec((_BLK, D), lambda i: (i, 0))
  col1 = pl.BlockSpec((_BLK, 1), lambda i: (i, 0))
  p0 = pl.BlockSpec((1, _BLK, D), lambda i: (0, i, 0))
  p1 = pl.BlockSpec((1, _BLK, D), lambda i: (1, i, 0))
  full = pl.BlockSpec((D, D), lambda i: (0, 0))
  bias = pl.BlockSpec((1, D), lambda i: (0, 0))
  proj = pl.BlockSpec((D, 4), lambda i: (0, 0))
  bias4 = pl.BlockSpec((1, 4), lambda i: (0, 0))
  return pl.pallas_call(
      body,
      grid=(_GRID,),
      in_specs=[row, p0, p1, col1, full, full, bias, proj, proj, bias4],
      out_specs=pl.BlockSpec((_BLK, 4), lambda i: (i, 0)),
      out_shape=jax.ShapeDtypeStruct((N, 4), jnp.float32),
  )(h1, aggp, aggp, rdeg, Wa, Wb, b, Wp1, Wp2, bp4)


@jax.jit
def kernel(x, edge_index, edge_weight, pred_edge_index, W1, b1, W2, b2, Wp, bp):
  ew = edge_weight.astype(jnp.float32)
  src = jnp.concatenate([edge_index[0], jnp.zeros((EPAD - E,), jnp.int32)])
  dst = jnp.concatenate([edge_index[1], jnp.zeros((EPAD - E,), jnp.int32)])
  w = jnp.concatenate([ew, jnp.zeros((EPAD - E,), jnp.float32)])
  e3 = jnp.stack([src.reshape(NW, NCH, CH), dst.reshape(NW, NCH, CH)],
                 axis=2)
  w3 = w.reshape(NW, NCH, CH)

  agg1p, degp = _make_seg(True)(x, e3, w3)
  deg0 = degp[0, :N, None]
  deg1 = degp[1, :N, None]
  h1, rdeg = _tc_layer1(x, agg1p, deg0, deg1, W1[:D], W1[D:], b1[None, :])

  agg2p = _make_seg(False)(h1, e3, w3)
  Wp1 = jnp.concatenate([Wp[0:D], Wp[2 * D:3 * D]], axis=1)
  Wp2 = jnp.concatenate([Wp[D:2 * D], Wp[3 * D:]], axis=1)
  bp4 = jnp.concatenate([bp, jnp.zeros((2,), jnp.float32)])[None, :]
  P = _tc_layer2(h1, agg2p, rdeg, W2[:D], W2[D:], b2[None, :], Wp1, Wp2, bp4)

  ps = jnp.concatenate([pred_edge_index[0],
                        jnp.zeros((PPAD - PE,), jnp.int32)]).reshape(NW, PCH, PW)
  pd = jnp.concatenate([pred_edge_index[1],
                        jnp.zeros((PPAD - PE,), jnp.int32)]).reshape(NW, PCH, PW)
  p0 = P[:, 0] + 0.0
  p1 = P[:, 1] + 0.0
  p2 = P[:, 2] + 0.0
  p3 = P[:, 3] + 0.0
  o0, o1 = _make_pred()(p0, p1, p2, p3, ps, pd)
  return jnp.stack([o0.reshape(-1)[:PE], o1.reshape(-1)[:PE]], axis=1)


# speedup vs baseline: 2.6238x; 1.4144x over previous
import jax
import jax.numpy as jnp
from jax import lax
from jax.experimental import pallas as pl
from jax.experimental.pallas import tpu as pltpu
from jax.experimental.pallas import tpu_sc as plsc

N = 10000
D = 128
E = 320000
PE = 100000

NC, NS = 2, 16
NW = NC * NS
CH = 112
NCH = 90
EPAD = NW * NCH * CH
NPAD = NS * 640
RPT = NPAD // NS
NB = 3
RD = 5

PCH = 25
PW = 128
PPAD = NW * PCH * PW

_mesh = lambda: plsc.VectorSubcoreMesh(core_axis_name="c", subcore_axis_name="s")


def _make_seg(want_deg):
  out_type = [jax.ShapeDtypeStruct((NC, NPAD, D), jnp.float32)]
  scratch = [
      pltpu.VMEM((RD, 2, CH), jnp.int32),
      pltpu.VMEM((RD, CH), jnp.float32),
      pltpu.VMEM((CH, D), jnp.float32),
      pltpu.VMEM((CH, D), jnp.float32),
      pltpu.VMEM((CH, D), jnp.float32),
      pltpu.VMEM_SHARED((NPAD, D), jnp.float32),
  ] + [pltpu.SemaphoreType.DMA] * 9
  if want_deg:
    out_type.append(jax.ShapeDtypeStruct((NC, NPAD), jnp.float32))
    scratch += [
        pltpu.VMEM((NB, CH), jnp.float32),
        pltpu.VMEM((RPT,), jnp.float32),
        pltpu.VMEM_SHARED((NPAD,), jnp.float32),
    ]

  def body(x_hbm, e_hbm, w_hbm, *rest):
    if want_deg:
      (out_hbm, deg_hbm, ed, wring, rows0, rows1, rows2, acc,
       gs0, gs1, gs2, ss0, ss1, ss2, fs0, fs1, fs2,
       abuf, zbuf, dacc) = rest
    else:
      (out_hbm, ed, wring, rows0, rows1, rows2, acc,
       gs0, gs1, gs2, ss0, ss1, ss2, fs0, fs1, fs2) = rest
      abuf = zbuf = dacc = None
    cid = lax.axis_index("c")
    sid = lax.axis_index("s")
    wid = cid * NS + sid
    rowsb = (rows0, rows1, rows2)
    gs = (gs0, gs1, gs2)
    ss = (ss0, ss1, ss2)
    fs = (fs0, fs1, fs2)

    def zrow(i, carry):
      for j in range(D // 16):
        rows0[i, pl.ds(j * 16, 16)] = jnp.zeros((16,), jnp.float32)
      return carry
    lax.fori_loop(0, CH, zrow, 0)
    for k in range(RPT // CH):
      pltpu.sync_copy(rows0, acc.at[pl.ds(sid * RPT + k * CH, CH)])
    if RPT % CH:
      pltpu.sync_copy(rows0.at[pl.ds(0, RPT % CH)],
                      acc.at[pl.ds(sid * RPT + (RPT // CH) * CH, RPT % CH)])
    if want_deg:
      def zd(i, carry):
        zbuf[pl.ds(i * 16, 16)] = jnp.zeros((16,), jnp.float32)
        return carry
      lax.fori_loop(0, RPT // 16, zd, 0)
      pltpu.sync_copy(zbuf, dacc.at[pl.ds(sid * RPT, RPT)])
    plsc.subcore_barrier()

    def slot(c):
      return c % RD if isinstance(c, int) else lax.rem(c, RD)

    def fill(c, s):
      pltpu.async_copy(e_hbm.at[wid, c], ed.at[slot(c)], fs[s])
      pltpu.async_copy(w_hbm.at[wid, c], wring.at[slot(c)], fs[s])

    def f_wait(c, s):
      pltpu.make_async_copy(e_hbm.at[wid, c], ed.at[slot(c)], fs[s]).wait()
      pltpu.make_async_copy(w_hbm.at[wid, c], wring.at[slot(c)],
                            fs[s]).wait()

    def g_issue(c, b):
      pltpu.async_copy(x_hbm.at[ed.at[slot(c), 0]], rowsb[b], gs[b])

    def g_wait(c, b):
      pltpu.make_async_copy(x_hbm.at[ed.at[slot(c), 0]], rowsb[b],
                            gs[b]).wait()

    def scale(c, b):
      rows = rowsb[b]
      r5 = slot(c)
      def egroup(g, carry):
        wr16 = wring[r5, pl.ds(g * 16, 16)]
        for r in range(16):
          ws = wr16[r]
          for j in range(D // 16):
            sl = pl.ds(j * 16, 16)
            rows[g * 16 + r, sl] = rows[g * 16 + r, sl] * ws
        return carry
      lax.fori_loop(0, CH // 16, egroup, 0)

    def s_issue(c, b):
      r5 = slot(c)
      pltpu.async_copy(rowsb[b], acc.at[ed.at[r5, 1]], ss[b], add=True)
      if want_deg:
        for j in range(CH // 16):
          sl = pl.ds(j * 16, 16)
          abuf[b, sl] = jnp.abs(wring[r5, sl])
        pltpu.async_copy(abuf.at[b], dacc.at[ed.at[r5, 1]], ss[b], add=True)

    def s_wait(c, b):
      r5 = slot(c)
      pltpu.make_async_copy(rowsb[b], acc.at[ed.at[r5, 1]], ss[b]).wait()
      if want_deg:
        pltpu.make_async_copy(abuf.at[b], dacc.at[ed.at[r5, 1]], ss[b]).wait()

    fill(0, 0)
    fill(1, 1)
    fill(2, 2)
    f_wait(0, 0)
    g_issue(0, 0)
    g_wait(0, 0); scale(0, 0); s_issue(0, 0)
    fill(3, 0)
    f_wait(1, 1); g_issue(1, 1)
    g_wait(1, 1); scale(1, 1); s_issue(1, 1)
    fill(4, 1)
    f_wait(2, 2); g_issue(2, 2)

    def step(i, carry):
      for b in range(3):
        c = 3 * i + b
        g_wait(c, b)
        scale(c, b)
        s_issue(c, b)
        s_wait(c - 2, (b + 1) % 3)
        @pl.when(c + 3 < NCH)
        def _():
          fill(c + 3, b)
        @pl.when(c + 1 < NCH)
        def _():
          f_wait(c + 1, (b + 1) % 3)
          g_issue(c + 1, (b + 1) % 3)
      return carry
    g_wait(2, 2); scale(2, 2); s_issue(2, 2)
    s_wait(0, 0)
    fill(5, 2)
    f_wait(3, 0); g_issue(3, 0)
    lax.fori_loop(1, NCH // 3, step, 0)

    s_wait(NCH - 2, (NCH - 2) % 3)
    s_wait(NCH - 1, (NCH - 1) % 3)
    plsc.subcore_barrier()

    pltpu.sync_copy(acc.at[pl.ds(sid * RPT, RPT)],
                    out_hbm.at[cid].at[pl.ds(sid * RPT, RPT)])
    if want_deg:
      pltpu.sync_copy(dacc.at[pl.ds(sid * RPT, RPT)],
                      deg_hbm.at[cid].at[pl.ds(sid * RPT, RPT)])

  return pl.kernel(body, out_type=tuple(out_type) if want_deg else out_type[0],
                   mesh=_mesh(), scratch_types=scratch)


def _make_pred():
  out_type = (jax.ShapeDtypeStruct((NW, PCH, PW), jnp.float32),
              jax.ShapeDtypeStruct((NW, PCH, PW), jnp.float32))
  scratch = [
      pltpu.VMEM((PCH, PW), jnp.int32),
      pltpu.VMEM((PCH, PW), jnp.int32),
      pltpu.VMEM((PW,), jnp.float32),
      pltpu.VMEM((PW,), jnp.float32),
      pltpu.VMEM((PW,), jnp.float32),
      pltpu.VMEM((PW,), jnp.float32),
      pltpu.SemaphoreType.DMA,
  ]

  def body(p0_hbm, p1_hbm, p2_hbm, p3_hbm, ps_hbm, pd_hbm,
           o0_hbm, o1_hbm, siv, div, a0, a1, b0, b1, sem):
    cid = lax.axis_index("c")
    sid = lax.axis_index("s")
    wid = cid * NS + sid
    pltpu.sync_copy(ps_hbm.at[wid], siv)
    pltpu.sync_copy(pd_hbm.at[wid], div)

    def chunk(c, carry):
      cp0 = pltpu.async_copy(p0_hbm.at[siv.at[c]], a0, sem)
      cp1 = pltpu.async_copy(p1_hbm.at[siv.at[c]], a1, sem)
      cp2 = pltpu.async_copy(p2_hbm.at[div.at[c]], b0, sem)
      cp3 = pltpu.async_copy(p3_hbm.at[div.at[c]], b1, sem)
      cp0.wait(); cp1.wait(); cp2.wait(); cp3.wait()
      for j in range(PW // 16):
        sl = pl.ds(j * 16, 16)
        a0[sl] = a0[sl] + b0[sl]
        a1[sl] = a1[sl] + b1[sl]
      pltpu.sync_copy(a0, o0_hbm.at[wid, c])
      pltpu.sync_copy(a1, o1_hbm.at[wid, c])
      return carry
    lax.fori_loop(0, PCH, chunk, 0)

  return pl.kernel(body, out_type=out_type, mesh=_mesh(),
                   scratch_types=scratch)


_BLK = 1000
_GRID = N // _BLK


def _tc_layer1(x, aggp, deg0, deg1, Wa, Wb, b):
  def body(x_r, a0_r, a1_r, d0_r, d1_r, wa_r, wb_r, b_r, h_r, rd_r):
    rd = 1.0 / jnp.maximum(d0_r[...] + d1_r[...], 1.0)
    a = (a0_r[0] + a1_r[0]) * rd
    acc = jnp.dot(x_r[...], wa_r[...], preferred_element_type=jnp.float32)
    acc += jnp.dot(a, wb_r[...], preferred_element_type=jnp.float32)
    h_r[...] = jnp.tanh(acc + b_r[...])
    rd_r[...] = rd

  row = pl.BlockSpec((_BLK, D), lambda i: (i, 0))
  col1 = pl.BlockSpec((_BLK, 1), lambda i: (i, 0))
  p0 = pl.BlockSpec((1, _BLK, D), lambda i: (0, i, 0))
  p1 = pl.BlockSpec((1, _BLK, D), lambda i: (1, i, 0))
  full = pl.BlockSpec((D, D), lambda i: (0, 0))
  bias = pl.BlockSpec((1, D), lambda i: (0, 0))
  return pl.pallas_call(
      body,
      grid=(_GRID,),
      in_specs=[row, p0, p1, col1, col1, full, full, bias],
      out_specs=[row, col1],
      out_shape=[jax.ShapeDtypeStruct((N, D), jnp.float32),
                 jax.ShapeDtypeStruct((N, 1), jnp.float32)],
  )(x, aggp, aggp, deg0, deg1, Wa, Wb, b)


def _tc_layer2(h1, aggp, rdeg, Wa, Wb, b, Wp1, Wp2, bp4):
  def body(h1_r, a0_r, a1_r, rd_r, wa_r, wb_r, b_r, wp1_r, wp2_r, bp_r, p_r):
    a = (a0_r[0] + a1_r[0]) * rd_r[...]
    acc = jnp.dot(h1_r[...], wa_r[...], preferred_element_type=jnp.float32)
    acc += jnp.dot(a, wb_r[...], preferred_element_type=jnp.float32)
    h2 = jnp.tanh(acc + b_r[...])
    p = jnp.dot(h1_r[...], wp1_r[...], preferred_element_type=jnp.float32)
    p += jnp.dot(h2, wp2_r[...], preferred_element_type=jnp.float32)
    p_r[...] = p + bp_r[...]

  row = pl.BlockSpec((_BLK, D), lambda i: (i, 0))
  col1 = pl.BlockSpec((_BLK, 1), lambda i: (i, 0))
  p0 = pl.BlockSpec((1, _BLK, D), lambda i: (0, i, 0))
  p1 = pl.BlockSpec((1, _BLK, D), lambda i: (1, i, 0))
  full = pl.BlockSpec((D, D), lambda i: (0, 0))
  bias = pl.BlockSpec((1, D), lambda i: (0, 0))
  proj = pl.BlockSpec((D, 4), lambda i: (0, 0))
  bias4 = pl.BlockSpec((1, 4), lambda i: (0, 0))
  return pl.pallas_call(
      body,
      grid=(_GRID,),
      in_specs=[row, p0, p1, col1, full, full, bias, proj, proj, bias4],
      out_specs=pl.BlockSpec((_BLK, 4), lambda i: (i, 0)),
      out_shape=jax.ShapeDtypeStruct((N, 4), jnp.float32),
  )(h1, aggp, aggp, rdeg, Wa, Wb, b, Wp1, Wp2, bp4)


@jax.jit
def kernel(x, edge_index, edge_weight, pred_edge_index, W1, b1, W2, b2, Wp, bp):
  ew = edge_weight.astype(jnp.float32)
  pad_idx = (jnp.arange(EPAD - E, dtype=jnp.int32) * 37) % N
  src = jnp.concatenate([edge_index[0], pad_idx])
  dst = jnp.concatenate([edge_index[1], pad_idx])
  w = jnp.concatenate([ew, jnp.zeros((EPAD - E,), jnp.float32)])
  e3 = jnp.stack([src.reshape(NW, NCH, CH), dst.reshape(NW, NCH, CH)],
                 axis=2)
  w3 = w.reshape(NW, NCH, CH)

  agg1p, degp = _make_seg(True)(x, e3, w3)
  deg0 = degp[0, :N, None]
  deg1 = degp[1, :N, None]
  h1, rdeg = _tc_layer1(x, agg1p, deg0, deg1, W1[:D], W1[D:], b1[None, :])

  agg2p = _make_seg(False)(h1, e3, w3)
  Wp1 = jnp.concatenate([Wp[0:D], Wp[2 * D:3 * D]], axis=1)
  Wp2 = jnp.concatenate([Wp[D:2 * D], Wp[3 * D:]], axis=1)
  bp4 = jnp.concatenate([bp, jnp.zeros((2,), jnp.float32)])[None, :]
  P = _tc_layer2(h1, agg2p, rdeg, W2[:D], W2[D:], b2[None, :], Wp1, Wp2, bp4)

  ps = jnp.concatenate([pred_edge_index[0],
                        jnp.zeros((PPAD - PE,), jnp.int32)]).reshape(NW, PCH, PW)
  pd = jnp.concatenate([pred_edge_index[1],
                        jnp.zeros((PPAD - PE,), jnp.int32)]).reshape(NW, PCH, PW)
  p0 = P[:, 0] + 0.0
  p1 = P[:, 1] + 0.0
  p2 = P[:, 2] + 0.0
  p3 = P[:, 3] + 0.0
  o0, o1 = _make_pred()(p0, p1, p2, p3, ps, pd)
  return jnp.stack([o0.reshape(-1)[:PE], o1.reshape(-1)[:PE]], axis=1)


# speedup vs baseline: 3.2052x; 1.2216x over previous
import jax
import jax.numpy as jnp
from jax import lax
from jax.experimental import pallas as pl
from jax.experimental.pallas import tpu as pltpu
from jax.experimental.pallas import tpu_sc as plsc

N = 10000
D = 128
E = 320000
PE = 100000

NC, NS = 2, 16
NW = NC * NS
CH = 112
NCH = 90
EPAD = NW * NCH * CH
NPAD = NS * 640
RPT = NPAD // NS
NB = 3
RD = 5

PCH = 25
PW = 128
PPAD = NW * PCH * PW

_mesh = lambda: plsc.VectorSubcoreMesh(core_axis_name="c", subcore_axis_name="s")


def _make_seg(want_deg):
  out_type = [jax.ShapeDtypeStruct((NC, NPAD, D), jnp.float32)]
  scratch = [
      pltpu.VMEM((RD, 2, CH), jnp.int32),
      pltpu.VMEM((RD, CH), jnp.float32),
      pltpu.VMEM((CH, D), jnp.float32),
      pltpu.VMEM((CH, D), jnp.float32),
      pltpu.VMEM((CH, D), jnp.float32),
      pltpu.VMEM_SHARED((NPAD, D), jnp.float32),
  ] + [pltpu.SemaphoreType.DMA] * 9
  if want_deg:
    out_type.append(jax.ShapeDtypeStruct((NC, NPAD), jnp.float32))
    scratch += [
        pltpu.VMEM((NB, CH), jnp.float32),
        pltpu.VMEM((RPT,), jnp.float32),
        pltpu.VMEM_SHARED((NPAD,), jnp.float32),
    ]

  def body(x_hbm, e_hbm, w_hbm, *rest):
    if want_deg:
      (out_hbm, deg_hbm, ed, wring, rows0, rows1, rows2, acc,
       gs0, gs1, gs2, ss0, ss1, ss2, fs0, fs1, fs2,
       abuf, zbuf, dacc) = rest
    else:
      (out_hbm, ed, wring, rows0, rows1, rows2, acc,
       gs0, gs1, gs2, ss0, ss1, ss2, fs0, fs1, fs2) = rest
      abuf = zbuf = dacc = None
    cid = lax.axis_index("c")
    sid = lax.axis_index("s")
    wid = cid * NS + sid
    rowsb = (rows0, rows1, rows2)
    gs = (gs0, gs1, gs2)
    ss = (ss0, ss1, ss2)
    fs = (fs0, fs1, fs2)

    def zrow(i, carry):
      for j in range(D // 16):
        rows0[i, pl.ds(j * 16, 16)] = jnp.zeros((16,), jnp.float32)
      return carry
    lax.fori_loop(0, CH, zrow, 0)
    for k in range(RPT // CH):
      pltpu.sync_copy(rows0, acc.at[pl.ds(sid * RPT + k * CH, CH)])
    if RPT % CH:
      pltpu.sync_copy(rows0.at[pl.ds(0, RPT % CH)],
                      acc.at[pl.ds(sid * RPT + (RPT // CH) * CH, RPT % CH)])
    if want_deg:
      def zd(i, carry):
        zbuf[pl.ds(i * 16, 16)] = jnp.zeros((16,), jnp.float32)
        return carry
      lax.fori_loop(0, RPT // 16, zd, 0)
      pltpu.sync_copy(zbuf, dacc.at[pl.ds(sid * RPT, RPT)])
    plsc.subcore_barrier()

    def slot(c):
      return c % RD if isinstance(c, int) else lax.rem(c, RD)

    def fill(c, s):
      pltpu.async_copy(e_hbm.at[wid, c], ed.at[slot(c)], fs[s])
      pltpu.async_copy(w_hbm.at[wid, c], wring.at[slot(c)], fs[s])

    def f_wait(c, s):
      pltpu.make_async_copy(e_hbm.at[wid, c], ed.at[slot(c)], fs[s]).wait()
      pltpu.make_async_copy(w_hbm.at[wid, c], wring.at[slot(c)],
                            fs[s]).wait()

    def g_issue(c, b):
      pltpu.async_copy(x_hbm.at[ed.at[slot(c), 0]], rowsb[b], gs[b])

    def g_wait(c, b):
      pltpu.make_async_copy(x_hbm.at[ed.at[slot(c), 0]], rowsb[b],
                            gs[b]).wait()

    def scale(c, b):
      rows = rowsb[b]
      r5 = slot(c)
      def egroup(g, carry):
        wr16 = wring[r5, pl.ds(g * 16, 16)]
        for r in range(16):
          ws = wr16[r]
          for j in range(D // 16):
            sl = pl.ds(j * 16, 16)
            rows[g * 16 + r, sl] = rows[g * 16 + r, sl] * ws
        return carry
      lax.fori_loop(0, CH // 16, egroup, 0)

    def s_issue(c, b):
      r5 = slot(c)
      pltpu.async_copy(rowsb[b], acc.at[ed.at[r5, 1]], ss[b], add=True)
      if want_deg:
        for j in range(CH // 16):
          sl = pl.ds(j * 16, 16)
          abuf[b, sl] = jnp.abs(wring[r5, sl])
        pltpu.async_copy(abuf.at[b], dacc.at[ed.at[r5, 1]], ss[b], add=True)

    def s_wait(c, b):
      r5 = slot(c)
      pltpu.make_async_copy(rowsb[b], acc.at[ed.at[r5, 1]], ss[b]).wait()
      if want_deg:
        pltpu.make_async_copy(abuf.at[b], dacc.at[ed.at[r5, 1]], ss[b]).wait()

    fill(0, 0)
    fill(1, 1)
    fill(2, 2)
    f_wait(0, 0)
    g_issue(0, 0)
    g_wait(0, 0)
    f_wait(1, 1); g_issue(1, 1)
    scale(0, 0); s_issue(0, 0)
    fill(3, 0)
    g_wait(1, 1)
    f_wait(2, 2); g_issue(2, 2)
    scale(1, 1); s_issue(1, 1)
    fill(4, 1)

    def step(i, carry):
      for b in range(3):
        c = 3 * i + b
        g_wait(c, b)
        s_wait(c - 2, (b + 1) % 3)
        @pl.when(c + 1 < NCH)
        def _():
          f_wait(c + 1, (b + 1) % 3)
          g_issue(c + 1, (b + 1) % 3)
        scale(c, b)
        s_issue(c, b)
        @pl.when(c + 3 < NCH)
        def _():
          fill(c + 3, b)
      return carry
    g_wait(2, 2)
    s_wait(0, 0)
    f_wait(3, 0); g_issue(3, 0)
    scale(2, 2); s_issue(2, 2)
    fill(5, 2)
    lax.fori_loop(1, NCH // 3, step, 0)

    s_wait(NCH - 2, (NCH - 2) % 3)
    s_wait(NCH - 1, (NCH - 1) % 3)
    plsc.subcore_barrier()

    pltpu.sync_copy(acc.at[pl.ds(sid * RPT, RPT)],
                    out_hbm.at[cid].at[pl.ds(sid * RPT, RPT)])
    if want_deg:
      pltpu.sync_copy(dacc.at[pl.ds(sid * RPT, RPT)],
                      deg_hbm.at[cid].at[pl.ds(sid * RPT, RPT)])

  return pl.kernel(body, out_type=tuple(out_type) if want_deg else out_type[0],
                   mesh=_mesh(), scratch_types=scratch)


def _make_pred():
  out_type = (jax.ShapeDtypeStruct((NW, PCH, PW), jnp.float32),
              jax.ShapeDtypeStruct((NW, PCH, PW), jnp.float32))
  scratch = [
      pltpu.VMEM((PCH, PW), jnp.int32),
      pltpu.VMEM((PCH, PW), jnp.int32),
      pltpu.VMEM((PW,), jnp.float32),
      pltpu.VMEM((PW,), jnp.float32),
      pltpu.VMEM((PW,), jnp.float32),
      pltpu.VMEM((PW,), jnp.float32),
      pltpu.SemaphoreType.DMA,
  ]

  def body(p0_hbm, p1_hbm, p2_hbm, p3_hbm, ps_hbm, pd_hbm,
           o0_hbm, o1_hbm, siv, div, a0, a1, b0, b1, sem):
    cid = lax.axis_index("c")
    sid = lax.axis_index("s")
    wid = cid * NS + sid
    pltpu.sync_copy(ps_hbm.at[wid], siv)
    pltpu.sync_copy(pd_hbm.at[wid], div)

    def chunk(c, carry):
      cp0 = pltpu.async_copy(p0_hbm.at[siv.at[c]], a0, sem)
      cp1 = pltpu.async_copy(p1_hbm.at[siv.at[c]], a1, sem)
      cp2 = pltpu.async_copy(p2_hbm.at[div.at[c]], b0, sem)
      cp3 = pltpu.async_copy(p3_hbm.at[div.at[c]], b1, sem)
      cp0.wait(); cp1.wait(); cp2.wait(); cp3.wait()
      for j in range(PW // 16):
        sl = pl.ds(j * 16, 16)
        a0[sl] = a0[sl] + b0[sl]
        a1[sl] = a1[sl] + b1[sl]
      pltpu.sync_copy(a0, o0_hbm.at[wid, c])
      pltpu.sync_copy(a1, o1_hbm.at[wid, c])
      return carry
    lax.fori_loop(0, PCH, chunk, 0)

  return pl.kernel(body, out_type=out_type, mesh=_mesh(),
                   scratch_types=scratch)


_BLK = 1000
_GRID = N // _BLK


def _tc_layer1(x, aggp, deg0, deg1, Wa, Wb, b):
  def body(x_r, a0_r, a1_r, d0_r, d1_r, wa_r, wb_r, b_r, h_r, rd_r):
    rd = 1.0 / jnp.maximum(d0_r[...] + d1_r[...], 1.0)
    a = (a0_r[0] + a1_r[0]) * rd
    acc = jnp.dot(x_r[...], wa_r[...], preferred_element_type=jnp.float32)
    acc += jnp.dot(a, wb_r[...], preferred_element_type=jnp.float32)
    h_r[...] = jnp.tanh(acc + b_r[...])
    rd_r[...] = rd

  row = pl.BlockSpec((_BLK, D), lambda i: (i, 0))
  col1 = pl.BlockSpec((_BLK, 1), lambda i: (i, 0))
  p0 = pl.BlockSpec((1, _BLK, D), lambda i: (0, i, 0))
  p1 = pl.BlockSpec((1, _BLK, D), lambda i: (1, i, 0))
  full = pl.BlockSpec((D, D), lambda i: (0, 0))
  bias = pl.BlockSpec((1, D), lambda i: (0, 0))
  return pl.pallas_call(
      body,
      grid=(_GRID,),
      in_specs=[row, p0, p1, col1, col1, full, full, bias],
      out_specs=[row, col1],
      out_shape=[jax.ShapeDtypeStruct((N, D), jnp.float32),
                 jax.ShapeDtypeStruct((N, 1), jnp.float32)],
  )(x, aggp, aggp, deg0, deg1, Wa, Wb, b)


def _tc_layer2(h1, aggp, rdeg, Wa, Wb, b, Wp1, Wp2, bp4):
  def body(h1_r, a0_r, a1_r, rd_r, wa_r, wb_r, b_r, wp1_r, wp2_r, bp_r, p_r):
    a = (a0_r[0] + a1_r[0]) * rd_r[...]
    acc = jnp.dot(h1_r[...], wa_r[...], preferred_element_type=jnp.float32)
    acc += jnp.dot(a, wb_r[...], preferred_element_type=jnp.float32)
    h2 = jnp.tanh(acc + b_r[...])
    p = jnp.dot(h1_r[...], wp1_r[...], preferred_element_type=jnp.float32)
    p += jnp.dot(h2, wp2_r[...], preferred_element_type=jnp.float32)
    p_r[...] = p + bp_r[...]

  row = pl.BlockSpec((_BLK, D), lambda i: (i, 0))
  col1 = pl.BlockSpec((_BLK, 1), lambda i: (i, 0))
  p0 = pl.BlockSpec((1, _BLK, D), lambda i: (0, i, 0))
  p1 = pl.BlockSpec((1, _BLK, D), lambda i: (1, i, 0))
  full = pl.BlockSpec((D, D), lambda i: (0, 0))
  bias = pl.BlockSpec((1, D), lambda i: (0, 0))
  proj = pl.BlockSpec((D, 4), lambda i: (0, 0))
  bias4 = pl.BlockSpec((1, 4), lambda i: (0, 0))
  return pl.pallas_call(
      body,
      grid=(_GRID,),
      in_specs=[row, p0, p1, col1, full, full, bias, proj, proj, bias4],
      out_specs=pl.BlockSpec((_BLK, 4), lambda i: (i, 0)),
      out_shape=jax.ShapeDtypeStruct((N, 4), jnp.float32),
  )(h1, aggp, aggp, rdeg, Wa, Wb, b, Wp1, Wp2, bp4)


@jax.jit
def kernel(x, edge_index, edge_weight, pred_edge_index, W1, b1, W2, b2, Wp, bp):
  ew = edge_weight.astype(jnp.float32)
  pad_idx = (jnp.arange(EPAD - E, dtype=jnp.int32) * 37) % N
  src = jnp.concatenate([edge_index[0], pad_idx])
  dst = jnp.concatenate([edge_index[1], pad_idx])
  w = jnp.concatenate([ew, jnp.zeros((EPAD - E,), jnp.float32)])
  e3 = jnp.stack([src.reshape(NW, NCH, CH), dst.reshape(NW, NCH, CH)],
                 axis=2)
  w3 = w.reshape(NW, NCH, CH)

  agg1p, degp = _make_seg(True)(x, e3, w3)
  deg0 = degp[0, :N, None]
  deg1 = degp[1, :N, None]
  h1, rdeg = _tc_layer1(x, agg1p, deg0, deg1, W1[:D], W1[D:], b1[None, :])

  agg2p = _make_seg(False)(h1, e3, w3)
  Wp1 = jnp.concatenate([Wp[0:D], Wp[2 * D:3 * D]], axis=1)
  Wp2 = jnp.concatenate([Wp[D:2 * D], Wp[3 * D:]], axis=1)
  bp4 = jnp.concatenate([bp, jnp.zeros((2,), jnp.float32)])[None, :]
  P = _tc_layer2(h1, agg2p, rdeg, W2[:D], W2[D:], b2[None, :], Wp1, Wp2, bp4)

  ps = jnp.concatenate([pred_edge_index[0],
                        jnp.zeros((PPAD - PE,), jnp.int32)]).reshape(NW, PCH, PW)
  pd = jnp.concatenate([pred_edge_index[1],
                        jnp.zeros((PPAD - PE,), jnp.int32)]).reshape(NW, PCH, PW)
  p0 = P[:, 0] + 0.0
  p1 = P[:, 1] + 0.0
  p2 = P[:, 2] + 0.0
  p3 = P[:, 3] + 0.0
  o0, o1 = _make_pred()(p0, p1, p2, p3, ps, pd)
  return jnp.stack([o0.reshape(-1)[:PE], o1.reshape(-1)[:PE]], axis=1)


# speedup vs baseline: 3.5065x; 1.0940x over previous
import jax
import jax.numpy as jnp
from jax import lax
from jax.experimental import pallas as pl
from jax.experimental.pallas import tpu as pltpu
from jax.experimental.pallas import tpu_sc as plsc

N = 10000
D = 128
E = 320000
PE = 100000

NC, NS = 2, 16
NW = NC * NS
CH = 112
NCH = 90
EPAD = NW * NCH * CH
NPAD = NS * 640
RPT = NPAD // NS
NB = 3
RD = 5

PCH = 25
PW = 128
PPAD = NW * PCH * PW

_mesh = lambda: plsc.VectorSubcoreMesh(core_axis_name="c", subcore_axis_name="s")


def _make_seg(want_deg):
  out_type = [jax.ShapeDtypeStruct((NC, NPAD, D), jnp.float32)]
  scratch = [
      pltpu.VMEM((RD, 2, CH), jnp.int32),
      pltpu.VMEM((RD, CH), jnp.float32),
      pltpu.VMEM((CH, D), jnp.float32),
      pltpu.VMEM((CH, D), jnp.float32),
      pltpu.VMEM((CH, D), jnp.float32),
      pltpu.VMEM_SHARED((NPAD, D), jnp.float32),
  ] + [pltpu.SemaphoreType.DMA] * 9
  if want_deg:
    out_type.append(jax.ShapeDtypeStruct((NC, NPAD), jnp.float32))
    scratch += [
        pltpu.VMEM((NB, CH), jnp.float32),
        pltpu.VMEM((RPT,), jnp.float32),
        pltpu.VMEM_SHARED((NPAD,), jnp.float32),
    ]

  def body(x_hbm, e_hbm, w_hbm, *rest):
    if want_deg:
      (out_hbm, deg_hbm, ed, wring, rows0, rows1, rows2, acc,
       gs0, gs1, gs2, ss0, ss1, ss2, fs0, fs1, fs2,
       abuf, zbuf, dacc) = rest
    else:
      (out_hbm, ed, wring, rows0, rows1, rows2, acc,
       gs0, gs1, gs2, ss0, ss1, ss2, fs0, fs1, fs2) = rest
      abuf = zbuf = dacc = None
    cid = lax.axis_index("c")
    sid = lax.axis_index("s")
    wid = cid * NS + sid
    rowsb = (rows0, rows1, rows2)
    gs = (gs0, gs1, gs2)
    ss = (ss0, ss1, ss2)
    fs = (fs0, fs1, fs2)

    def zrow(i, carry):
      for j in range(D // 16):
        rows0[i, pl.ds(j * 16, 16)] = jnp.zeros((16,), jnp.float32)
      return carry
    lax.fori_loop(0, CH, zrow, 0)
    for k in range(RPT // CH):
      pltpu.sync_copy(rows0, acc.at[pl.ds(sid * RPT + k * CH, CH)])
    if RPT % CH:
      pltpu.sync_copy(rows0.at[pl.ds(0, RPT % CH)],
                      acc.at[pl.ds(sid * RPT + (RPT // CH) * CH, RPT % CH)])
    if want_deg:
      def zd(i, carry):
        zbuf[pl.ds(i * 16, 16)] = jnp.zeros((16,), jnp.float32)
        return carry
      lax.fori_loop(0, RPT // 16, zd, 0)
      pltpu.sync_copy(zbuf, dacc.at[pl.ds(sid * RPT, RPT)])
    plsc.subcore_barrier()

    def slot(c):
      return c % RD if isinstance(c, int) else lax.rem(c, RD)

    def fill(c, s):
      pltpu.async_copy(e_hbm.at[wid, c], ed.at[slot(c)], fs[s])
      pltpu.async_copy(w_hbm.at[wid, c], wring.at[slot(c)], fs[s])

    def f_wait(c, s):
      pltpu.make_async_copy(e_hbm.at[wid, c], ed.at[slot(c)], fs[s]).wait()
      pltpu.make_async_copy(w_hbm.at[wid, c], wring.at[slot(c)],
                            fs[s]).wait()

    def g_issue(c, b):
      pltpu.async_copy(x_hbm.at[ed.at[slot(c), 0]], rowsb[b], gs[b])

    def g_wait(c, b):
      pltpu.make_async_copy(x_hbm.at[ed.at[slot(c), 0]], rowsb[b],
                            gs[b]).wait()

    def scale(c, b):
      rows = rowsb[b]
      r5 = slot(c)
      def egroup(g, carry):
        wr16 = wring[r5, pl.ds(g * 16, 16)]
        for r in range(16):
          ws = wr16[r]
          for j in range(D // 16):
            sl = pl.ds(j * 16, 16)
            rows[g * 16 + r, sl] = rows[g * 16 + r, sl] * ws
        return carry
      lax.fori_loop(0, CH // 16, egroup, 0)

    def s_issue(c, b):
      r5 = slot(c)
      pltpu.async_copy(rowsb[b], acc.at[ed.at[r5, 1]], ss[b], add=True)
      if want_deg:
        for j in range(CH // 16):
          sl = pl.ds(j * 16, 16)
          abuf[b, sl] = jnp.abs(wring[r5, sl])
        pltpu.async_copy(abuf.at[b], dacc.at[ed.at[r5, 1]], ss[b], add=True)

    def s_wait(c, b):
      r5 = slot(c)
      pltpu.make_async_copy(rowsb[b], acc.at[ed.at[r5, 1]], ss[b]).wait()
      if want_deg:
        pltpu.make_async_copy(abuf.at[b], dacc.at[ed.at[r5, 1]], ss[b]).wait()

    fill(0, 0)
    fill(1, 1)
    fill(2, 2)
    f_wait(0, 0)
    g_issue(0, 0)
    g_wait(0, 0)
    f_wait(1, 1); g_issue(1, 1)
    scale(0, 0); s_issue(0, 0)
    fill(3, 0)
    g_wait(1, 1)
    f_wait(2, 2); g_issue(2, 2)
    scale(1, 1); s_issue(1, 1)
    s_wait(0, 0)
    fill(4, 1)

    def step(i, carry):
      for b in range(3):
        c = 3 * i + b
        g_wait(c, b)
        scale(c, b)
        s_issue(c, b)
        s_wait(c - 1, (b + 2) % 3)
        @pl.when(c + 4 < NCH)
        def _():
          fill(c + 4, (b + 1) % 3)
        @pl.when(c + 2 < NCH)
        def _():
          f_wait(c + 2, (b + 2) % 3)
          g_issue(c + 2, (b + 2) % 3)
      return carry
    g_wait(2, 2)
    f_wait(3, 0); g_issue(3, 0)
    scale(2, 2); s_issue(2, 2)
    s_wait(1, 1)
    fill(5, 2)
    fill(6, 0)
    f_wait(4, 1); g_issue(4, 1)
    lax.fori_loop(1, NCH // 3, step, 0)

    s_wait(NCH - 1, (NCH - 1) % 3)
    plsc.subcore_barrier()

    pltpu.sync_copy(acc.at[pl.ds(sid * RPT, RPT)],
                    out_hbm.at[cid].at[pl.ds(sid * RPT, RPT)])
    if want_deg:
      pltpu.sync_copy(dacc.at[pl.ds(sid * RPT, RPT)],
                      deg_hbm.at[cid].at[pl.ds(sid * RPT, RPT)])

  return pl.kernel(body, out_type=tuple(out_type) if want_deg else out_type[0],
                   mesh=_mesh(), scratch_types=scratch)


def _make_pred():
  out_type = (jax.ShapeDtypeStruct((NW, PCH, PW), jnp.float32),
              jax.ShapeDtypeStruct((NW, PCH, PW), jnp.float32))
  scratch = [
      pltpu.VMEM((PCH, PW), jnp.int32),
      pltpu.VMEM((PCH, PW), jnp.int32),
      pltpu.VMEM((PW,), jnp.float32),
      pltpu.VMEM((PW,), jnp.float32),
      pltpu.VMEM((PW,), jnp.float32),
      pltpu.VMEM((PW,), jnp.float32),
      pltpu.SemaphoreType.DMA,
  ]

  def body(p0_hbm, p1_hbm, p2_hbm, p3_hbm, ps_hbm, pd_hbm,
           o0_hbm, o1_hbm, siv, div, a0, a1, b0, b1, sem):
    cid = lax.axis_index("c")
    sid = lax.axis_index("s")
    wid = cid * NS + sid
    pltpu.sync_copy(ps_hbm.at[wid], siv)
    pltpu.sync_copy(pd_hbm.at[wid], div)

    def chunk(c, carry):
      cp0 = pltpu.async_copy(p0_hbm.at[siv.at[c]], a0, sem)
      cp1 = pltpu.async_copy(p1_hbm.at[siv.at[c]], a1, sem)
      cp2 = pltpu.async_copy(p2_hbm.at[div.at[c]], b0, sem)
      cp3 = pltpu.async_copy(p3_hbm.at[div.at[c]], b1, sem)
      cp0.wait(); cp1.wait(); cp2.wait(); cp3.wait()
      for j in range(PW // 16):
        sl = pl.ds(j * 16, 16)
        a0[sl] = a0[sl] + b0[sl]
        a1[sl] = a1[sl] + b1[sl]
      pltpu.sync_copy(a0, o0_hbm.at[wid, c])
      pltpu.sync_copy(a1, o1_hbm.at[wid, c])
      return carry
    lax.fori_loop(0, PCH, chunk, 0)

  return pl.kernel(body, out_type=out_type, mesh=_mesh(),
                   scratch_types=scratch)


_BLK = 1000
_GRID = N // _BLK


def _tc_layer1(x, aggp, deg0, deg1, Wa, Wb, b):
  def body(x_r, a0_r, a1_r, d0_r, d1_r, wa_r, wb_r, b_r, h_r, rd_r):
    rd = 1.0 / jnp.maximum(d0_r[...] + d1_r[...], 1.0)
    a = (a0_r[0] + a1_r[0]) * rd
    acc = jnp.dot(x_r[...], wa_r[...], preferred_element_type=jnp.float32)
    acc += jnp.dot(a, wb_r[...], preferred_element_type=jnp.float32)
    h_r[...] = jnp.tanh(acc + b_r[...])
    rd_r[...] = rd

  row = pl.BlockSpec((_BLK, D), lambda i: (i, 0))
  col1 = pl.BlockSpec((_BLK, 1), lambda i: (i, 0))
  p0 = pl.BlockSpec((1, _BLK, D), lambda i: (0, i, 0))
  p1 = pl.BlockSpec((1, _BLK, D), lambda i: (1, i, 0))
  full = pl.BlockSpec((D, D), lambda i: (0, 0))
  bias = pl.BlockSpec((1, D), lambda i: (0, 0))
  return pl.pallas_call(
      body,
      grid=(_GRID,),
      in_specs=[row, p0, p1, col1, col1, full, full, bias],
      out_specs=[row, col1],
      out_shape=[jax.ShapeDtypeStruct((N, D), jnp.float32),
                 jax.ShapeDtypeStruct((N, 1), jnp.float32)],
  )(x, aggp, aggp, deg0, deg1, Wa, Wb, b)


def _tc_layer2(h1, aggp, rdeg, Wa, Wb, b, Wp1, Wp2, bp4):
  def body(h1_r, a0_r, a1_r, rd_r, wa_r, wb_r, b_r, wp1_r, wp2_r, bp_r, p_r):
    a = (a0_r[0] + a1_r[0]) * rd_r[...]
    acc = jnp.dot(h1_r[...], wa_r[...], preferred_element_type=jnp.float32)
    acc += jnp.dot(a, wb_r[...], preferred_element_type=jnp.float32)
    h2 = jnp.tanh(acc + b_r[...])
    p = jnp.dot(h1_r[...], wp1_r[...], preferred_element_type=jnp.float32)
    p += jnp.dot(h2, wp2_r[...], preferred_element_type=jnp.float32)
    p_r[...] = p + bp_r[...]

  row = pl.BlockSpec((_BLK, D), lambda i: (i, 0))
  col1 = pl.BlockSpec((_BLK, 1), lambda i: (i, 0))
  p0 = pl.BlockSpec((1, _BLK, D), lambda i: (0, i, 0))
  p1 = pl.BlockSpec((1, _BLK, D), lambda i: (1, i, 0))
  full = pl.BlockSpec((D, D), lambda i: (0, 0))
  bias = pl.BlockSpec((1, D), lambda i: (0, 0))
  proj = pl.BlockSpec((D, 4), lambda i: (0, 0))
  bias4 = pl.BlockSpec((1, 4), lambda i: (0, 0))
  return pl.pallas_call(
      body,
      grid=(_GRID,),
      in_specs=[row, p0, p1, col1, full, full, bias, proj, proj, bias4],
      out_specs=pl.BlockSpec((_BLK, 4), lambda i: (i, 0)),
      out_shape=jax.ShapeDtypeStruct((N, 4), jnp.float32),
  )(h1, aggp, aggp, rdeg, Wa, Wb, b, Wp1, Wp2, bp4)


@jax.jit
def kernel(x, edge_index, edge_weight, pred_edge_index, W1, b1, W2, b2, Wp, bp):
  ew = edge_weight.astype(jnp.float32)
  pad_idx = (jnp.arange(EPAD - E, dtype=jnp.int32) * 37) % N
  src = jnp.concatenate([edge_index[0], pad_idx])
  dst = jnp.concatenate([edge_index[1], pad_idx])
  w = jnp.concatenate([ew, jnp.zeros((EPAD - E,), jnp.float32)])
  e3 = jnp.stack([src.reshape(NW, NCH, CH), dst.reshape(NW, NCH, CH)],
                 axis=2)
  w3 = w.reshape(NW, NCH, CH)

  agg1p, degp = _make_seg(True)(x, e3, w3)
  deg0 = degp[0, :N, None]
  deg1 = degp[1, :N, None]
  h1, rdeg = _tc_layer1(x, agg1p, deg0, deg1, W1[:D], W1[D:], b1[None, :])

  agg2p = _make_seg(False)(h1, e3, w3)
  Wp1 = jnp.concatenate([Wp[0:D], Wp[2 * D:3 * D]], axis=1)
  Wp2 = jnp.concatenate([Wp[D:2 * D], Wp[3 * D:]], axis=1)
  bp4 = jnp.concatenate([bp, jnp.zeros((2,), jnp.float32)])[None, :]
  P = _tc_layer2(h1, agg2p, rdeg, W2[:D], W2[D:], b2[None, :], Wp1, Wp2, bp4)

  ps = jnp.concatenate([pred_edge_index[0],
                        jnp.zeros((PPAD - PE,), jnp.int32)]).reshape(NW, PCH, PW)
  pd = jnp.concatenate([pred_edge_index[1],
                        jnp.zeros((PPAD - PE,), jnp.int32)]).reshape(NW, PCH, PW)
  p0 = P[:, 0] + 0.0
  p1 = P[:, 1] + 0.0
  p2 = P[:, 2] + 0.0
  p3 = P[:, 3] + 0.0
  o0, o1 = _make_pred()(p0, p1, p2, p3, ps, pd)
  return jnp.stack([o0.reshape(-1)[:PE], o1.reshape(-1)[:PE]], axis=1)
